# baseline (speedup 1.0000x reference)
import dataclasses

import jax
import jax.numpy as jnp
from jax import lax
from jax.experimental import pallas as pl
from jax.experimental.pallas import tpu as pltpu
from jax.experimental.pallas import tpu_sc as plsc

NC = 2
NS = 16
NW = NC * NS
L = 16


def _compiler_params():
    cp = pltpu.CompilerParams(use_tc_tiling_on_sc=False)
    if "needs_layout_passes" in pltpu.CompilerParams.__dataclass_fields__:
        cp = dataclasses.replace(cp, needs_layout_passes=False)
    return cp


def _gather_call(batch, seq, emb, dtype):
    mesh = plsc.VectorSubcoreMesh(core_axis_name="c", subcore_axis_name="s")
    bt_n = batch // 128
    dt_n = emb // 8
    units = seq * bt_n
    per_w = units // NW

    @jax.jit
    def run(table, flat_idx):
        @pl.kernel(
            out_type=jax.ShapeDtypeStruct((seq, dt_n, bt_n, 8, 128), dtype),
            mesh=mesh,
            compiler_params=_compiler_params(),
            scratch_types=[
                pltpu.VMEM((128,), jnp.int32),
                pltpu.VMEM((128, emb), dtype),
                pltpu.VMEM((emb, 128), dtype),
                pltpu.SemaphoreType.DMA,
            ],
        )
        def kern(table_hbm, idx_hbm, out_hbm, idx_v, rows_v, trans_v, sem):
            wid = lax.axis_index("s") * NC + lax.axis_index("c")
            i16 = lax.iota(jnp.int32, L)

            @pl.loop(0, per_w)
            def _(g):
                u = wid * per_w + g
                t = u // bt_n
                b = u % bt_n
                base = t * batch + b * 128
                pltpu.sync_copy(idx_hbm.at[0, pl.ds(base, 128)], idx_v)
                pltpu.async_copy(table_hbm.at[idx_v], rows_v, sem).wait()
                @pl.loop(0, L)
                def _(j):
                    perm = lax.rem(i16 + j, L)
                    for l0 in range(0, 128, L):
                        src_l = i16 + l0
                        for d0 in range(0, emb, L):
                            src_d = perm + d0
                            v = plsc.load_gather(rows_v, [src_l, src_d])
                            plsc.store_scatter(trans_v, [src_d, src_l], v)
                for d in range(dt_n):
                    pltpu.sync_copy(
                        trans_v.at[pl.ds(8 * d, 8)], out_hbm.at[t, d, b]
                    )

        return kern(table, flat_idx)

    return run


def kernel(input_sentence, table):
    batch, seq = input_sentence.shape
    vocab, emb = table.shape
    flat_idx = input_sentence.T.reshape(1, batch * seq).astype(jnp.int32)
    run = _gather_call(batch, seq, emb, table.dtype)
    out5 = run(table, flat_idx)
    out = out5.transpose(2, 4, 0, 1, 3).reshape(batch, seq, emb)
    return out


# speedup vs baseline: 1.4164x; 1.4164x over previous
import dataclasses

import jax
import jax.numpy as jnp
from jax import lax
from jax.experimental import pallas as pl
from jax.experimental.pallas import tpu as pltpu
from jax.experimental.pallas import tpu_sc as plsc

NC = 2
NS = 16
NW = NC * NS
L = 16


def _compiler_params():
    cp = pltpu.CompilerParams(
        use_tc_tiling_on_sc=False, disable_bounds_checks=True
    )
    if "needs_layout_passes" in pltpu.CompilerParams.__dataclass_fields__:
        cp = dataclasses.replace(cp, needs_layout_passes=False)
    return cp


def _gather_call(batch, seq, emb, dtype):
    mesh = plsc.VectorSubcoreMesh(core_axis_name="c", subcore_axis_name="s")
    bt_n = batch // 128
    dt_n = emb // 8
    units = seq * bt_n
    per_w = units // NW
    idx_per_w = per_w * 128

    @jax.jit
    def run(table, flat_idx):
        @pl.kernel(
            out_type=jax.ShapeDtypeStruct((seq, dt_n, bt_n, 8, 128), dtype),
            mesh=mesh,
            compiler_params=_compiler_params(),
            scratch_types=[
                pltpu.VMEM((idx_per_w,), jnp.int32),
                pltpu.VMEM((2, 128, emb), dtype),
                pltpu.VMEM((2, emb, 128), dtype),
                pltpu.SemaphoreType.DMA,
                pltpu.SemaphoreType.DMA,
                pltpu.SemaphoreType.DMA,
                pltpu.SemaphoreType.DMA,
            ],
        )
        def kern(table_hbm, idx_hbm, out_hbm, idx_v, rows_v, trans_v,
                 sem_g0, sem_g1, sem_w0, sem_w1):
            wid = lax.axis_index("s") * NC + lax.axis_index("c")
            i16 = lax.iota(jnp.int32, L)
            u0 = wid * per_w
            gsems = (sem_g0, sem_g1)
            wsems = (sem_w0, sem_w1)

            pltpu.sync_copy(
                idx_hbm.at[0, pl.ds(u0 * 128, idx_per_w)], idx_v
            )

            def issue_gather(g, p):
                pltpu.async_copy(
                    table_hbm.at[idx_v.at[pl.ds(g * 128, 128)]],
                    rows_v.at[p],
                    gsems[p],
                )

            def wait_gather(p):
                pltpu.make_async_copy(
                    table_hbm.at[idx_v.at[pl.ds(0, 128)]],
                    rows_v.at[p],
                    gsems[p],
                ).wait()

            def drain_writes(p):
                for d in range(dt_n):
                    pltpu.make_async_copy(
                        trans_v.at[p, pl.ds(8 * d, 8)],
                        out_hbm.at[0, d, 0],
                        wsems[p],
                    ).wait()

            def transpose(p):
                @pl.loop(0, L)
                def _(j):
                    perm = lax.rem(i16 + j, L)
                    for l0 in range(0, 128, L):
                        src_l = i16 + l0
                        for d0 in range(0, emb, L):
                            src_d = perm + d0
                            v = plsc.load_gather(
                                rows_v.at[p], [src_l, src_d]
                            )
                            plsc.store_scatter(
                                trans_v.at[p], [src_d, src_l], v
                            )

            def fire_writes(g, p):
                u = u0 + g
                t = u // bt_n
                b = u % bt_n
                for d in range(dt_n):
                    pltpu.async_copy(
                        trans_v.at[p, pl.ds(8 * d, 8)],
                        out_hbm.at[t, d, b],
                        wsems[p],
                    )

            issue_gather(0, 0)

            @pl.loop(0, per_w, step=2)
            def _(g):
                issue_gather(g + 1, 1)
                wait_gather(0)

                @pl.when(g >= 2)
                def _():
                    drain_writes(0)

                transpose(0)
                fire_writes(g, 0)

                @pl.when(g + 2 < per_w)
                def _():
                    issue_gather(g + 2, 0)

                wait_gather(1)

                @pl.when(g >= 2)
                def _():
                    drain_writes(1)

                transpose(1)
                fire_writes(g + 1, 1)

            drain_writes(0)
            drain_writes(1)

        return kern(table, flat_idx)

    return run


def kernel(input_sentence, table):
    batch, seq = input_sentence.shape
    vocab, emb = table.shape
    flat_idx = input_sentence.T.reshape(1, batch * seq).astype(jnp.int32)
    run = _gather_call(batch, seq, emb, table.dtype)
    out5 = run(table, flat_idx)
    out = out5.transpose(2, 4, 0, 1, 3).reshape(batch, seq, emb)
    return out


# speedup vs baseline: 1.4195x; 1.0022x over previous
import dataclasses

import jax
import jax.numpy as jnp
from jax import lax
from jax.experimental import pallas as pl
from jax.experimental.pallas import tpu as pltpu
from jax.experimental.pallas import tpu_sc as plsc

NC = 2
NS = 16
NW = NC * NS
L = 16


def _compiler_params():
    cp = pltpu.CompilerParams(
        use_tc_tiling_on_sc=False, disable_bounds_checks=True
    )
    if "needs_layout_passes" in pltpu.CompilerParams.__dataclass_fields__:
        cp = dataclasses.replace(cp, needs_layout_passes=False)
    return cp


def _gather_call(batch, seq, emb, dtype):
    mesh = plsc.VectorSubcoreMesh(core_axis_name="c", subcore_axis_name="s")
    bt_n = batch // 128
    dt_n = emb // 8
    units = seq * bt_n
    per_w = units // NW
    idx_per_w = per_w * 128

    @jax.jit
    def run(table, flat_idx):
        @pl.kernel(
            out_type=jax.ShapeDtypeStruct((seq, dt_n, bt_n, 8, 128), dtype),
            mesh=mesh,
            compiler_params=_compiler_params(),
            scratch_types=[
                pltpu.VMEM((idx_per_w,), jnp.int32),
                pltpu.VMEM((2, 128, emb), dtype),
                pltpu.VMEM((2, emb, 128), dtype),
                pltpu.SemaphoreType.DMA,
                pltpu.SemaphoreType.DMA,
                pltpu.SemaphoreType.DMA,
                pltpu.SemaphoreType.DMA,
            ],
        )
        def kern(table_hbm, idx_hbm, out_hbm, idx_v, rows_v, trans_v,
                 sem_g0, sem_g1, sem_w0, sem_w1):
            wid = lax.axis_index("s") * NC + lax.axis_index("c")
            i16 = lax.iota(jnp.int32, L)
            u0 = wid * per_w
            gsems = (sem_g0, sem_g1)
            wsems = (sem_w0, sem_w1)

            pltpu.sync_copy(
                idx_hbm.at[0, pl.ds(u0 * 128, idx_per_w)], idx_v
            )

            def issue_gather(g, p):
                pltpu.async_copy(
                    table_hbm.at[idx_v.at[pl.ds(g * 128, 128)]],
                    rows_v.at[p],
                    gsems[p],
                )

            def wait_gather(p):
                pltpu.make_async_copy(
                    table_hbm.at[idx_v.at[pl.ds(0, 128)]],
                    rows_v.at[p],
                    gsems[p],
                ).wait()

            def drain_writes(p):
                for d in range(dt_n):
                    pltpu.make_async_copy(
                        trans_v.at[p, pl.ds(8 * d, 8)],
                        out_hbm.at[0, d, 0],
                        wsems[p],
                    ).wait()

            def transpose(p):
                @pl.loop(0, L)
                def _(j):
                    perm = lax.rem(i16 + j, L)
                    src_ds = [perm + d0 for d0 in range(0, emb, L)]
                    src_ls = [i16 + l0 for l0 in range(0, 128, L)]
                    for src_l in src_ls:
                        for src_d in src_ds:
                            v = plsc.load_gather(
                                rows_v.at[p], [src_l, src_d]
                            )
                            plsc.store_scatter(
                                trans_v.at[p], [src_d, src_l], v
                            )

            def fire_writes(g, p):
                u = u0 + g
                t = u // bt_n
                b = u % bt_n
                for d in range(dt_n):
                    pltpu.async_copy(
                        trans_v.at[p, pl.ds(8 * d, 8)],
                        out_hbm.at[t, d, b],
                        wsems[p],
                    )

            issue_gather(0, 0)

            @pl.loop(0, per_w, step=2)
            def _(g):
                issue_gather(g + 1, 1)
                wait_gather(0)

                @pl.when(g >= 2)
                def _():
                    drain_writes(0)

                transpose(0)
                fire_writes(g, 0)

                @pl.when(g + 2 < per_w)
                def _():
                    issue_gather(g + 2, 0)

                wait_gather(1)

                @pl.when(g >= 2)
                def _():
                    drain_writes(1)

                transpose(1)
                fire_writes(g + 1, 1)

            drain_writes(0)
            drain_writes(1)

        return kern(table, flat_idx)

    return run


def kernel(input_sentence, table):
    batch, seq = input_sentence.shape
    vocab, emb = table.shape
    flat_idx = input_sentence.T.reshape(1, batch * seq).astype(jnp.int32)
    run = _gather_call(batch, seq, emb, table.dtype)
    out5 = run(table, flat_idx)
    out = out5.transpose(2, 4, 0, 1, 3).reshape(batch, seq, emb)
    return out
